# baseline (speedup 1.0000x reference)
import functools

import jax
import jax.numpy as jnp
import numpy as np
from jax.experimental import pallas as pl
from jax.experimental.pallas import tpu as pltpu

KNN = 16
DM = 512
NLEVELS = 4


def _seq_sum(x3):
    acc = x3[:, 0, :]
    for k in range(1, x3.shape[1]):
        acc = acc + x3[:, k, :]
    return acc


def _split3(v):
    mask = np.uint32(0xFFFF0000)
    u = jax.lax.bitcast_convert_type(v, jnp.uint32)
    hi = jax.lax.bitcast_convert_type(u & mask, jnp.float32)
    r = v - hi
    ur = jax.lax.bitcast_convert_type(r, jnp.uint32)
    mid = jax.lax.bitcast_convert_type(ur & mask, jnp.float32)
    lo = r - mid
    return (hi.astype(jnp.bfloat16), mid.astype(jnp.bfloat16),
            lo.astype(jnp.bfloat16))


def _onehot_gather(oh, hi_ref, mid_ref, lo_ref):
    g = jnp.dot(oh, hi_ref[0], preferred_element_type=jnp.float32)
    g = g + jnp.dot(oh, mid_ref[0], preferred_element_type=jnp.float32)
    return g + jnp.dot(oh, lo_ref[0], preferred_element_type=jnp.float32)


def _tb_block_kernel(q_ref, idx_ref, khi_ref, kmid_ref, klo_ref,
                     vhi_ref, vmid_ref, vlo_ref,
                     xyz_ref, xhi_ref, xmid_ref, xlo_ref,
                     d1w_ref, d1b_ref, d2w_ref, d2b_ref,
                     g1w_ref, g1b_ref, g2w_ref, g2b_ref,
                     out_ref, *, m_tile):
    n, dm = khi_ref.shape[1], khi_ref.shape[2]
    mk = idx_ref.shape[1]
    k = mk // m_tile
    idx = idx_ref[0]
    oh = jnp.where(jax.lax.broadcasted_iota(jnp.int32, (mk, n), 1) == idx,
                   1.0, 0.0).astype(jnp.bfloat16)
    xyz_t = xyz_ref[0]
    kxyz = _onehot_gather(oh, xhi_ref, xmid_ref, xlo_ref)
    xyz_rep = jnp.broadcast_to(
        xyz_t[:, None, :], (m_tile, k, xyz_t.shape[1])).reshape(mk, -1)
    rel = xyz_rep - kxyz
    ph = jnp.maximum(
        jnp.dot(rel, d1w_ref[...], preferred_element_type=jnp.float32)
        + d1b_ref[...], 0.0)
    pos = d2b_ref[...] + jnp.dot(ph, d2w_ref[...],
                                 preferred_element_type=jnp.float32)
    q = q_ref[0]
    q_rep = jnp.broadcast_to(q[:, None, :], (m_tile, k, dm)).reshape(mk, dm)
    kg = _onehot_gather(oh, khi_ref, kmid_ref, klo_ref)
    gin = q_rep - kg + pos
    ah = jnp.maximum(
        g1b_ref[...]
        + jnp.dot(gin, g1w_ref[...], preferred_element_type=jnp.float32),
        0.0)
    attn = g2b_ref[...] + jnp.dot(ah, g2w_ref[...],
                                  preferred_element_type=jnp.float32)
    attn = attn / np.sqrt(dm)
    a3 = attn.reshape(m_tile, k, dm)
    amax = jnp.max(a3, axis=1, keepdims=True)
    e = jnp.exp(a3 - amax)
    p = e / _seq_sum(e)[:, None, :]
    vg = _onehot_gather(oh, vhi_ref, vmid_ref, vlo_ref)
    vp = (vg + pos).reshape(m_tile, k, dm)
    out_ref[0] = _seq_sum(p * vp)


def _row2(v):
    return v.reshape(1, -1)


def _transformer_block(p, xyz, feats, knn_idx):
    b, n, dp = feats.shape
    x = feats @ p['fc1_w'] + p['fc1_b']
    q = x @ p['wq']
    kk = x @ p['wk']
    vv = x @ p['wv']
    k = knn_idx.shape[2]
    m = min(n, 128)
    nt = n // m
    mk = m * k
    khi, kmid, klo = _split3(kk)
    vhi, vmid, vlo = _split3(vv)
    xhi, xmid, xlo = _split3(xyz)

    def bcast(shape):
        nd = len(shape)
        return pl.BlockSpec(shape, lambda bi, mi: (0,) * nd)

    tbl = pl.BlockSpec((1, n, DM), lambda bi, mi: (bi, 0, 0))
    tbl3 = pl.BlockSpec((1, n, 3), lambda bi, mi: (bi, 0, 0))
    res = pl.pallas_call(
        functools.partial(_tb_block_kernel, m_tile=m),
        grid=(b, nt),
        in_specs=[
            pl.BlockSpec((1, m, DM), lambda bi, mi: (bi, mi, 0)),
            pl.BlockSpec((1, mk, 1), lambda bi, mi: (bi, mi, 0)),
            tbl, tbl, tbl, tbl, tbl, tbl,
            pl.BlockSpec((1, m, 3), lambda bi, mi: (bi, mi, 0)),
            tbl3, tbl3, tbl3,
            bcast((3, DM)), bcast((1, DM)),
            bcast((DM, DM)), bcast((1, DM)),
            bcast((DM, DM)), bcast((1, DM)),
            bcast((DM, DM)), bcast((1, DM)),
        ],
        out_specs=pl.BlockSpec((1, m, DM), lambda bi, mi: (bi, mi, 0)),
        out_shape=jax.ShapeDtypeStruct((b, n, DM), jnp.float32),
        compiler_params=pltpu.CompilerParams(
            dimension_semantics=("parallel", "arbitrary")),
    )(q, knn_idx.reshape(b, n * k, 1), khi, kmid, klo, vhi, vmid, vlo,
      xyz, xhi, xmid, xlo,
      p['d1_w'], _row2(p['d1_b']), p['d2_w'], _row2(p['d2_b']),
      p['g1_w'], _row2(p['g1_b']), p['g2_w'], _row2(p['g2_b']))
    return res @ p['fc2_w'] + p['fc2_b'] + feats


def _sq_dist(src, dst):
    d = -2.0 * jnp.matmul(src, jnp.swapaxes(dst, -1, -2))
    d = d + jnp.sum(src ** 2, -1)[..., :, None]
    d = d + jnp.sum(dst ** 2, -1)[..., None, :]
    return d


def _knn_kernel(srct_ref, dstT_ref, srcn_ref, dstn_ref, out_ref, *, kk):
    d = -2.0 * jnp.dot(srct_ref[0], dstT_ref[0],
                       preferred_element_type=jnp.float32)
    d = d + srcn_ref[0]
    d = d + dstn_ref[0]
    iota = jax.lax.broadcasted_iota(jnp.int32, d.shape, 1)
    big = jnp.int32(2 ** 30)
    cols = []
    work = d
    for _ in range(kk):
        mn = jnp.min(work, axis=1, keepdims=True)
        cand = jnp.where(work == mn, iota, big)
        j = jnp.min(cand, axis=1, keepdims=True)
        cols.append(j)
        work = jnp.where(iota == j, jnp.float32(jnp.inf), work)
    out_ref[0] = jnp.concatenate(cols, axis=1)


def _knn_idx(src, dst, k):
    b, ns, _ = src.shape
    nd = dst.shape[1]
    k = min(k, nd)
    dst_t = jnp.swapaxes(dst, -1, -2)
    srcn = jnp.sum(src ** 2, -1)[..., :, None]
    dstn = jnp.sum(dst ** 2, -1)[..., None, :]
    ms = min(ns, 128)
    return pl.pallas_call(
        functools.partial(_knn_kernel, kk=k),
        grid=(b, ns // ms),
        in_specs=[
            pl.BlockSpec((1, ms, 3), lambda bi, mi: (bi, mi, 0)),
            pl.BlockSpec((1, 3, nd), lambda bi, mi: (bi, 0, 0)),
            pl.BlockSpec((1, ms, 1), lambda bi, mi: (bi, mi, 0)),
            pl.BlockSpec((1, 1, nd), lambda bi, mi: (bi, 0, 0)),
        ],
        out_specs=pl.BlockSpec((1, ms, k), lambda bi, mi: (bi, mi, 0)),
        out_shape=jax.ShapeDtypeStruct((b, ns, k), jnp.int32),
        compiler_params=pltpu.CompilerParams(
            dimension_semantics=("parallel", "arbitrary")),
    )(src, dst_t, srcn, dstn)


def _fps_kernel(xyzT_ref, out_ref, *, npoint):
    xx = xyzT_ref[0, 0:1, :]
    yy = xyzT_ref[0, 1:2, :]
    zz = xyzT_ref[0, 2:3, :]
    iota = jax.lax.broadcasted_iota(jnp.int32, xx.shape, 1)
    big = jnp.int32(2 ** 30)

    def sel(v, j):
        return jnp.sum(jnp.where(iota == j, v, 0.0), axis=1, keepdims=True)

    def body(t, carry):
        distance, far = carry
        out_ref[0, pl.ds(t, 1), :] = far
        cx, cy, cz = sel(xx, far), sel(yy, far), sel(zz, far)
        dist = (xx - cx) ** 2 + (yy - cy) ** 2 + (zz - cz) ** 2
        distance = jnp.minimum(distance, dist)
        mx = jnp.max(distance, axis=1, keepdims=True)
        nf = jnp.min(jnp.where(distance == mx, iota, big), axis=1,
                     keepdims=True)
        return distance, nf

    init = (jnp.full(xx.shape, 1e10, jnp.float32),
            jnp.zeros((1, 1), jnp.int32))
    jax.lax.fori_loop(0, npoint, body, init)


def _fps(xyz, npoint):
    b, n, _ = xyz.shape
    xyz_t = jnp.swapaxes(xyz, -1, -2)
    out = pl.pallas_call(
        functools.partial(_fps_kernel, npoint=npoint),
        grid=(b,),
        in_specs=[pl.BlockSpec((1, 3, n), lambda bi: (bi, 0, 0))],
        out_specs=pl.BlockSpec((1, npoint, 1), lambda bi: (bi, 0, 0)),
        out_shape=jax.ShapeDtypeStruct((b, npoint, 1), jnp.int32),
        compiler_params=pltpu.CompilerParams(
            dimension_semantics=("parallel",)),
    )(xyz_t)
    return out[..., 0]


def _set_abstraction(mlps, xyz, points, npoint):
    fps_idx = _fps(xyz, npoint)
    gather = jax.vmap(lambda a, i: a[i])
    new_xyz = gather(xyz, fps_idx)
    idx = _knn_idx(new_xyz, xyz, KNN)
    grouped_xyz = gather(xyz, idx)
    grouped_xyz_norm = grouped_xyz - new_xyz[:, :, None, :]
    grouped_points = gather(points, idx)
    h = jnp.concatenate([grouped_xyz_norm, grouped_points], axis=-1)
    for layer in mlps:
        h = h @ layer['w'].T + layer['b']
        mean = jnp.mean(h, axis=(0, 1, 2), keepdims=True)
        var = jnp.var(h, axis=(0, 1, 2), keepdims=True)
        h = (h - mean) / jnp.sqrt(var + 1e-5) * layer['gamma'] + layer['beta']
        h = jax.nn.relu(h)
    new_points = jnp.max(h, axis=2)
    return new_xyz, new_points


def kernel(x, params):
    xyz = x[..., :3]
    f = jax.nn.relu(x @ params['fc1_w1'] + params['fc1_b1'])
    f = f @ params['fc1_w2'] + params['fc1_b2']
    points = _transformer_block(params['tb0'], xyz, f,
                                _knn_idx(xyz, xyz, KNN))
    xyz_and_feats = [(xyz, points)]
    for i in range(NLEVELS):
        npoint = 1024 // 4 ** (i + 1)
        blk = params['blocks'][i]
        xyz, points = _set_abstraction(blk['sa_mlps'], xyz, points, npoint)
        points = _transformer_block(blk['tb'], xyz, points,
                                    _knn_idx(xyz, xyz, KNN))
        xyz_and_feats.append((xyz, points))
    return points, tuple(xyz_and_feats)


# speedup vs baseline: 1.0261x; 1.0261x over previous
import functools

import jax
import jax.numpy as jnp
import numpy as np
from jax.experimental import pallas as pl
from jax.experimental.pallas import tpu as pltpu

KNN = 16
DM = 512
NLEVELS = 4


def _seq_sum(x3):
    acc = x3[:, 0, :]
    for k in range(1, x3.shape[1]):
        acc = acc + x3[:, k, :]
    return acc


def _split3(v):
    mask = np.uint32(0xFFFF0000)
    u = jax.lax.bitcast_convert_type(v, jnp.uint32)
    hi = jax.lax.bitcast_convert_type(u & mask, jnp.float32)
    r = v - hi
    ur = jax.lax.bitcast_convert_type(r, jnp.uint32)
    mid = jax.lax.bitcast_convert_type(ur & mask, jnp.float32)
    lo = r - mid
    return (hi.astype(jnp.bfloat16), mid.astype(jnp.bfloat16),
            lo.astype(jnp.bfloat16))


def _onehot_gather(oh, hi_ref, mid_ref, lo_ref):
    g = jnp.dot(oh, hi_ref[0], preferred_element_type=jnp.float32)
    g = g + jnp.dot(oh, mid_ref[0], preferred_element_type=jnp.float32)
    return g + jnp.dot(oh, lo_ref[0], preferred_element_type=jnp.float32)


def _tb_block_kernel(q_ref, idx_ref, khi_ref, kmid_ref, klo_ref,
                     vhi_ref, vmid_ref, vlo_ref,
                     xyz_ref, xhi_ref, xmid_ref, xlo_ref,
                     d1w_ref, d1b_ref, d2w_ref, d2b_ref,
                     g1w_ref, g1b_ref, g2w_ref, g2b_ref,
                     out_ref, *, m_tile):
    n, dm = khi_ref.shape[1], khi_ref.shape[2]
    mk = idx_ref.shape[1]
    k = mk // m_tile
    idx = idx_ref[0]
    oh = jnp.where(jax.lax.broadcasted_iota(jnp.int32, (mk, n), 1) == idx,
                   1.0, 0.0).astype(jnp.bfloat16)
    xyz_t = xyz_ref[0]
    kxyz = _onehot_gather(oh, xhi_ref, xmid_ref, xlo_ref)
    xyz_rep = jnp.broadcast_to(
        xyz_t[:, None, :], (m_tile, k, xyz_t.shape[1])).reshape(mk, -1)
    rel = xyz_rep - kxyz
    ph = jnp.maximum(
        jnp.dot(rel, d1w_ref[...], preferred_element_type=jnp.float32)
        + d1b_ref[...], 0.0)
    pos = d2b_ref[...] + jnp.dot(ph, d2w_ref[...],
                                 preferred_element_type=jnp.float32)
    q = q_ref[0]
    q_rep = jnp.broadcast_to(q[:, None, :], (m_tile, k, dm)).reshape(mk, dm)
    kg = _onehot_gather(oh, khi_ref, kmid_ref, klo_ref)
    gin = q_rep - kg + pos
    ah = jnp.maximum(
        g1b_ref[...]
        + jnp.dot(gin, g1w_ref[...], preferred_element_type=jnp.float32),
        0.0)
    attn = g2b_ref[...] + jnp.dot(ah, g2w_ref[...],
                                  preferred_element_type=jnp.float32)
    attn = attn / np.sqrt(dm)
    a3 = attn.reshape(m_tile, k, dm)
    amax = jnp.max(a3, axis=1, keepdims=True)
    e = jnp.exp(a3 - amax)
    p = e / _seq_sum(e)[:, None, :]
    vg = _onehot_gather(oh, vhi_ref, vmid_ref, vlo_ref)
    vp = (vg + pos).reshape(m_tile, k, dm)
    out_ref[0] = _seq_sum(p * vp)


def _row2(v):
    return v.reshape(1, -1)


def _transformer_block(p, xyz, feats, knn_idx):
    b, n, dp = feats.shape
    x = feats @ p['fc1_w'] + p['fc1_b']
    q = x @ p['wq']
    kk = x @ p['wk']
    vv = x @ p['wv']
    k = knn_idx.shape[2]
    m = min(n, 128)
    nt = n // m
    mk = m * k
    khi, kmid, klo = _split3(kk)
    vhi, vmid, vlo = _split3(vv)
    xhi, xmid, xlo = _split3(xyz)

    def bcast(shape):
        nd = len(shape)
        return pl.BlockSpec(shape, lambda bi, mi: (0,) * nd)

    tbl = pl.BlockSpec((1, n, DM), lambda bi, mi: (bi, 0, 0))
    tbl3 = pl.BlockSpec((1, n, 3), lambda bi, mi: (bi, 0, 0))
    res = pl.pallas_call(
        functools.partial(_tb_block_kernel, m_tile=m),
        grid=(b, nt),
        in_specs=[
            pl.BlockSpec((1, m, DM), lambda bi, mi: (bi, mi, 0)),
            pl.BlockSpec((1, mk, 1), lambda bi, mi: (bi, mi, 0)),
            tbl, tbl, tbl, tbl, tbl, tbl,
            pl.BlockSpec((1, m, 3), lambda bi, mi: (bi, mi, 0)),
            tbl3, tbl3, tbl3,
            bcast((3, DM)), bcast((1, DM)),
            bcast((DM, DM)), bcast((1, DM)),
            bcast((DM, DM)), bcast((1, DM)),
            bcast((DM, DM)), bcast((1, DM)),
        ],
        out_specs=pl.BlockSpec((1, m, DM), lambda bi, mi: (bi, mi, 0)),
        out_shape=jax.ShapeDtypeStruct((b, n, DM), jnp.float32),
        compiler_params=pltpu.CompilerParams(
            dimension_semantics=("parallel", "arbitrary")),
    )(q, knn_idx.reshape(b, n * k, 1), khi, kmid, klo, vhi, vmid, vlo,
      xyz, xhi, xmid, xlo,
      p['d1_w'], _row2(p['d1_b']), p['d2_w'], _row2(p['d2_b']),
      p['g1_w'], _row2(p['g1_b']), p['g2_w'], _row2(p['g2_b']))
    return res @ p['fc2_w'] + p['fc2_b'] + feats


def _sq_dist(src, dst):
    d = -2.0 * jnp.matmul(src, jnp.swapaxes(dst, -1, -2))
    d = d + jnp.sum(src ** 2, -1)[..., :, None]
    d = d + jnp.sum(dst ** 2, -1)[..., None, :]
    return d


def _knn_kernel(srct_ref, dstT_ref, srcn_ref, dstn_ref, out_ref, *, kk):
    d = -2.0 * jnp.dot(srct_ref[0], dstT_ref[0],
                       preferred_element_type=jnp.float32)
    d = d + srcn_ref[0]
    d = d + dstn_ref[0]
    iota = jax.lax.broadcasted_iota(jnp.int32, d.shape, 1)
    big = jnp.int32(2 ** 30)
    cols = []
    work = d
    for _ in range(kk):
        mn = jnp.min(work, axis=1, keepdims=True)
        cand = jnp.where(work == mn, iota, big)
        j = jnp.min(cand, axis=1, keepdims=True)
        cols.append(j)
        work = jnp.where(iota == j, jnp.float32(jnp.inf), work)
    out_ref[0] = jnp.concatenate(cols, axis=1)


def _knn_idx(src, dst, k):
    b, ns, _ = src.shape
    nd = dst.shape[1]
    k = min(k, nd)
    dst_t = jnp.swapaxes(dst, -1, -2)
    srcn = jnp.sum(src ** 2, -1)[..., :, None]
    dstn = jnp.sum(dst ** 2, -1)[..., None, :]
    ms = min(ns, 128)
    return pl.pallas_call(
        functools.partial(_knn_kernel, kk=k),
        grid=(b, ns // ms),
        in_specs=[
            pl.BlockSpec((1, ms, 3), lambda bi, mi: (bi, mi, 0)),
            pl.BlockSpec((1, 3, nd), lambda bi, mi: (bi, 0, 0)),
            pl.BlockSpec((1, ms, 1), lambda bi, mi: (bi, mi, 0)),
            pl.BlockSpec((1, 1, nd), lambda bi, mi: (bi, 0, 0)),
        ],
        out_specs=pl.BlockSpec((1, ms, k), lambda bi, mi: (bi, mi, 0)),
        out_shape=jax.ShapeDtypeStruct((b, ns, k), jnp.int32),
        compiler_params=pltpu.CompilerParams(
            dimension_semantics=("parallel", "arbitrary")),
    )(src, dst_t, srcn, dstn)


def _fps(xyz, npoint):
    b, n, _ = xyz.shape

    def body(carry, _):
        distance, farthest = carry
        centroid = jax.vmap(lambda p, i: p[i])(xyz, farthest)
        dist = jnp.sum((xyz - centroid[:, None, :]) ** 2, -1)
        distance = jnp.minimum(distance, dist)
        new_farthest = jnp.argmax(distance, -1).astype(jnp.int32)
        return (distance, new_farthest), farthest

    init = (jnp.full((b, n), 1e10, jnp.float32), jnp.zeros((b,), jnp.int32))
    _, cent = jax.lax.scan(body, init, None, length=npoint)
    return jnp.transpose(cent)


def _set_abstraction(mlps, xyz, points, npoint):
    fps_idx = _fps(xyz, npoint)
    gather = jax.vmap(lambda a, i: a[i])
    new_xyz = gather(xyz, fps_idx)
    idx = _knn_idx(new_xyz, xyz, KNN)
    grouped_xyz = gather(xyz, idx)
    grouped_xyz_norm = grouped_xyz - new_xyz[:, :, None, :]
    grouped_points = gather(points, idx)
    h = jnp.concatenate([grouped_xyz_norm, grouped_points], axis=-1)
    for layer in mlps:
        h = h @ layer['w'].T + layer['b']
        mean = jnp.mean(h, axis=(0, 1, 2), keepdims=True)
        var = jnp.var(h, axis=(0, 1, 2), keepdims=True)
        h = (h - mean) / jnp.sqrt(var + 1e-5) * layer['gamma'] + layer['beta']
        h = jax.nn.relu(h)
    new_points = jnp.max(h, axis=2)
    return new_xyz, new_points


def kernel(x, params):
    xyz = x[..., :3]
    f = jax.nn.relu(x @ params['fc1_w1'] + params['fc1_b1'])
    f = f @ params['fc1_w2'] + params['fc1_b2']
    points = _transformer_block(params['tb0'], xyz, f,
                                _knn_idx(xyz, xyz, KNN))
    xyz_and_feats = [(xyz, points)]
    for i in range(NLEVELS):
        npoint = 1024 // 4 ** (i + 1)
        blk = params['blocks'][i]
        xyz, points = _set_abstraction(blk['sa_mlps'], xyz, points, npoint)
        points = _transformer_block(blk['tb'], xyz, points,
                                    _knn_idx(xyz, xyz, KNN))
        xyz_and_feats.append((xyz, points))
    return points, tuple(xyz_and_feats)
